# baseline (speedup 1.0000x reference)
import functools

import jax
import jax.numpy as jnp
from jax import lax
from jax.experimental import pallas as pl
from jax.experimental.pallas import tpu as pltpu
from jax.experimental.pallas import tpu_sc as plsc

N_ROWS = 512
N_CODES = 8192
DIM = 32
CHUNK = 1024
N_CHUNKS = N_CODES // CHUNK


def _argmin_kernel(xf_ref, et_ref, idx_ref, best_ref, bidx_ref):
    s = pl.program_id(0)
    xm = xf_ref[:, :]
    ec = et_ref[:, :]
    d = None
    for r in range(4):
        t = []
        for i in range(8):
            k = 8 * r + i
            dd = xm[:, k:k + 1] - ec[k:k + 1, :]
            t.append(dd * dd)
        b0 = t[0] + t[4]
        b1 = t[1] + t[5]
        b2 = t[2] + t[6]
        b3 = t[3] + t[7]
        sgrp = (b0 + b2) + (b1 + b3)
        d = sgrp if d is None else d + sgrp
    m = jnp.min(d, axis=1, keepdims=True)
    iota = jax.lax.broadcasted_iota(jnp.int32, (N_ROWS, CHUNK), 1)
    masked = jnp.where(d == m, iota, N_CODES)
    a = jnp.min(masked, axis=1, keepdims=True) + s * CHUNK

    @pl.when(s == 0)
    def _init():
        best_ref[:, :] = m
        bidx_ref[:, :] = a

    @pl.when(s > 0)
    def _update():
        prev_m = best_ref[:, :]
        prev_a = bidx_ref[:, :]
        upd = m < prev_m
        best_ref[:, :] = jnp.where(upd, m, prev_m)
        bidx_ref[:, :] = jnp.where(upd, a, prev_a)

    @pl.when(s == N_CHUNKS - 1)
    def _emit():
        idx_ref[:, :] = bidx_ref[:, :]


def _tc_argmin(xf, et):
    return pl.pallas_call(
        _argmin_kernel,
        grid=(N_CHUNKS,),
        in_specs=[
            pl.BlockSpec((N_ROWS, DIM), lambda s: (0, 0)),
            pl.BlockSpec((DIM, CHUNK), lambda s: (0, s)),
        ],
        out_specs=pl.BlockSpec((N_ROWS, 1), lambda s: (0, 0)),
        out_shape=jax.ShapeDtypeStruct((N_ROWS, 1), jnp.int32),
        scratch_shapes=[
            pltpu.VMEM((N_ROWS, 1), jnp.float32),
            pltpu.VMEM((N_ROWS, 1), jnp.int32),
        ],
    )(xf, et)


D_PAD = 128


def _make_sc_gather():
    info = plsc.get_sparse_core_info()
    nw = info.num_cores * info.num_subcores
    b_per_w = N_ROWS // nw
    mesh = plsc.VectorSubcoreMesh(core_axis_name="c", subcore_axis_name="s")

    @functools.partial(
        pl.kernel, mesh=mesh,
        out_type=jax.ShapeDtypeStruct((N_ROWS, D_PAD), jnp.float32),
        scratch_types=[
            pltpu.VMEM((b_per_w,), jnp.int32),
            pltpu.VMEM((b_per_w, D_PAD), jnp.float32),
            pltpu.SemaphoreType.DMA,
        ],
    )
    def sc_gather(table_hbm, idx_hbm, out_hbm, idx_v, rows_v, sem):
        wid = lax.axis_index("s") * info.num_cores + lax.axis_index("c")
        base = wid * b_per_w
        pltpu.sync_copy(idx_hbm.at[pl.ds(base, b_per_w)], idx_v)
        pltpu.async_copy(table_hbm.at[idx_v], rows_v, sem).wait()
        pltpu.sync_copy(rows_v, out_hbm.at[pl.ds(base, b_per_w)])

    return sc_gather


_sc_gather = _make_sc_gather()


@jax.jit
def kernel(x, embed_weight):
    ori_shape = x.shape
    b, ch, h, w = ori_shape
    xf = jnp.transpose(x, (0, 2, 3, 1)).reshape(b * h * w, ch)
    et = embed_weight.T

    idx = _tc_argmin(xf, et).reshape(N_ROWS)
    e_pad = jnp.pad(embed_weight, ((0, 0), (0, D_PAD - DIM)))
    q = _sc_gather(e_pad, idx)[:, :DIM]
    return q.reshape(ori_shape)
